# Initial kernel scaffold; baseline (speedup 1.0000x reference)
import jax
import jax.numpy as jnp
from jax.experimental import pallas as pl


def kernel(x, edge_index, W, att_src, att_dst, bias):
    raise NotImplementedError("write your pallas kernel here")


# speedup vs baseline: 13.8318x; 13.8318x over previous
import functools

import jax
import jax.numpy as jnp
from jax import lax
from jax.experimental import pallas as pl
from jax.experimental.pallas import tpu as pltpu
from jax.experimental.pallas import tpu_sc as plsc

N = 10000
E = 320000
C = 128
C2 = C // 2

NPAD = 10240
NC, NS, L = 2, 16, 16
EP1 = E // NS
EPT = E // (NC * NS)
CH = 80
BC = 2000
RPT = NPAD // NS


def _prep_body(x_ref, w_ref, asv_ref, adv_ref,
               xwa_ref, xwb_ref, asrc_ref, adst_ref, m_ref):
    xw = jnp.dot(x_ref[...], w_ref[...], preferred_element_type=jnp.float32)
    xwa_ref[...] = xw[:, :C2]
    xwb_ref[...] = xw[:, C2:]
    a_src = jnp.sum(xw * asv_ref[...], axis=1)
    a_dst = jnp.sum(xw * adv_ref[...], axis=1)
    asrc_ref[...] = a_src[None, :]
    adst_ref[...] = a_dst[None, :]
    m = jnp.max(a_src) + jnp.max(a_dst)
    m = jnp.where(m >= 0.0, m, 0.2 * m)
    m_ref[...] = jnp.full((1, 128), m, jnp.float32)


def _finish_body(p_ref, b_ref, o_ref):
    o_ref[:, :C2] = p_ref[0, 0] + p_ref[1, 0] + b_ref[:, :C2]
    o_ref[:, C2:] = p_ref[0, 1] + p_ref[1, 1] + b_ref[:, C2:]


_sc_mesh = plsc.VectorSubcoreMesh(core_axis_name="c", subcore_axis_name="s")


@functools.partial(
    pl.kernel,
    out_type=jax.ShapeDtypeStruct((NC, 2, NPAD, C2), jnp.float32),
    mesh=_sc_mesh,
    compiler_params=pltpu.CompilerParams(needs_layout_passes=False, use_tc_tiling_on_sc=False),
    scratch_types=[
        pltpu.VMEM((N,), jnp.float32),
        pltpu.VMEM((N,), jnp.float32),
        pltpu.VMEM((1, 128), jnp.float32),
        pltpu.VMEM((BC,), jnp.int32),
        pltpu.VMEM((BC,), jnp.int32),
        pltpu.VMEM((NPAD,), jnp.float32),
        pltpu.VMEM((CH,), jnp.float32),
        pltpu.VMEM((CH,), jnp.int32),
        pltpu.VMEM((CH,), jnp.int32),
        pltpu.VMEM((CH,), jnp.float32),
        pltpu.VMEM((CH, C2), jnp.float32),
        pltpu.VMEM((RPT,), jnp.float32),
        pltpu.VMEM_SHARED((NPAD,), jnp.float32),
        pltpu.VMEM_SHARED((NPAD, C2), jnp.float32),
    ],
)
def _sc_gat(xwa_hbm, xwb_hbm, asrc_hbm, adst_hbm, m_hbm, src_hbm, dst_hbm,
            pout_hbm,
            asrc_v, adst_v, m_v, src_c, dst_c, den_v,
            ee_sm, dst_sm, src_sm, al_sm, rows_v, zb_v, den_sh, out_sh):
    c = lax.axis_index("c")
    s = lax.axis_index("s")

    pltpu.sync_copy(asrc_hbm.at[0], asrc_v)
    pltpu.sync_copy(adst_hbm.at[0], adst_v)
    pltpu.sync_copy(m_hbm, m_v)
    mvec = m_v[0, pl.ds(0, L)]

    def _zb(i, carry):
        zb_v[pl.ds(i * L, L)] = jnp.zeros((L,), jnp.float32)
        return carry
    lax.fori_loop(0, RPT // L, _zb, 0)

    def _zr(i, carry):
        for q in range(C2 // L):
            rows_v[i, pl.ds(q * L, L)] = jnp.zeros((L,), jnp.float32)
        return carry
    lax.fori_loop(0, CH, _zr, 0)

    pltpu.sync_copy(zb_v, den_sh.at[pl.ds(s * RPT, RPT)])

    def _zo(j, carry):
        pltpu.sync_copy(rows_v, out_sh.at[pl.ds(s * RPT + j * CH, CH)])
        return carry
    lax.fori_loop(0, RPT // CH, _zo, 0)

    plsc.subcore_barrier()

    def _p1(j, carry):
        base = s * EP1 + j * BC
        pltpu.sync_copy(src_hbm.at[pl.ds(base, BC)], src_c)
        pltpu.sync_copy(dst_hbm.at[pl.ds(base, BC)], dst_c)

        def _p1c(t, cc):
            for v in range(CH // L):
                o = t * CH + v * L
                si = src_c[pl.ds(o, L)]
                di = dst_c[pl.ds(o, L)]
                e = plsc.load_gather(asrc_v, [si]) + plsc.load_gather(adst_v, [di])
                e = jnp.where(e >= 0.0, e, 0.2 * e)
                ee_sm[pl.ds(v * L, L)] = jnp.exp(e - mvec)
                dst_sm[pl.ds(v * L, L)] = di
            pltpu.sync_copy(ee_sm, den_sh.at[dst_sm], add=True)
            return cc
        lax.fori_loop(0, BC // CH, _p1c, 0)
        return carry
    lax.fori_loop(0, EP1 // BC, _p1, 0)

    plsc.subcore_barrier()
    pltpu.sync_copy(den_sh, den_v)

    for h, xw_hbm in enumerate((xwa_hbm, xwb_hbm)):
        if h == 1:
            plsc.subcore_barrier()
            def _zr2(i, carry):
                for q in range(C2 // L):
                    rows_v[i, pl.ds(q * L, L)] = jnp.zeros((L,), jnp.float32)
                return carry
            lax.fori_loop(0, CH, _zr2, 0)
            def _zo2(j, carry):
                pltpu.sync_copy(rows_v, out_sh.at[pl.ds(s * RPT + j * CH, CH)])
                return carry
            lax.fori_loop(0, RPT // CH, _zo2, 0)
            plsc.subcore_barrier()

        def _p2(u, carry):
            base = s * EP1 + c * EPT + u * BC
            pltpu.sync_copy(src_hbm.at[pl.ds(base, BC)], src_c)
            pltpu.sync_copy(dst_hbm.at[pl.ds(base, BC)], dst_c)

            def _p2c(t, cc):
                for v in range(CH // L):
                    o = t * CH + v * L
                    src_sm[pl.ds(v * L, L)] = src_c[pl.ds(o, L)]
                    dst_sm[pl.ds(v * L, L)] = dst_c[pl.ds(o, L)]
                pltpu.sync_copy(xw_hbm.at[src_sm], rows_v)
                for v in range(CH // L):
                    si = src_sm[pl.ds(v * L, L)]
                    di = dst_sm[pl.ds(v * L, L)]
                    e = (plsc.load_gather(asrc_v, [si])
                         + plsc.load_gather(adst_v, [di]))
                    e = jnp.where(e >= 0.0, e, 0.2 * e)
                    ee = jnp.exp(e - mvec)
                    den = plsc.load_gather(den_v, [di])
                    al_sm[pl.ds(v * L, L)] = ee / (den + 1e-16)

                def _scale(g, dd):
                    av = al_sm[pl.ds(g * L, L)]
                    for j in range(L):
                        a = av[j]
                        r = g * L + j
                        for q in range(C2 // L):
                            rows_v[r, pl.ds(q * L, L)] = (
                                rows_v[r, pl.ds(q * L, L)] * a)
                    return dd
                lax.fori_loop(0, CH // L, _scale, 0)
                pltpu.sync_copy(rows_v, out_sh.at[dst_sm], add=True)
                return cc
            lax.fori_loop(0, BC // CH, _p2c, 0)
            return carry
        lax.fori_loop(0, EPT // BC, _p2, 0)

        plsc.subcore_barrier()
        pltpu.sync_copy(out_sh.at[pl.ds(s * RPT, RPT)],
                        pout_hbm.at[c, h, pl.ds(s * RPT, RPT)])


def kernel(x, edge_index, W, att_src, att_dst, bias):
    src = edge_index[0].astype(jnp.int32)
    dst = edge_index[1].astype(jnp.int32)
    asv = att_src.reshape(1, C)
    adv = att_dst.reshape(1, C)

    xwa, xwb, asrc, adst, m = pl.pallas_call(
        _prep_body,
        out_shape=[
            jax.ShapeDtypeStruct((N, C2), jnp.float32),
            jax.ShapeDtypeStruct((N, C2), jnp.float32),
            jax.ShapeDtypeStruct((1, N), jnp.float32),
            jax.ShapeDtypeStruct((1, N), jnp.float32),
            jax.ShapeDtypeStruct((1, 128), jnp.float32),
        ],
    )(x, W, asv, adv)

    pout = _sc_gat(xwa, xwb, asrc, adst, m, src, dst)

    out = pl.pallas_call(
        _finish_body,
        out_shape=jax.ShapeDtypeStruct((NPAD, C), jnp.float32),
    )(pout, bias.reshape(1, C))
    return out[:N]


# speedup vs baseline: 18.2187x; 1.3172x over previous
import functools

import jax
import jax.numpy as jnp
from jax import lax
from jax.experimental import pallas as pl
from jax.experimental.pallas import tpu as pltpu
from jax.experimental.pallas import tpu_sc as plsc

N = 10000
E = 320000
C = 128
C2 = C // 2

NPAD = 10240
NC, NS, L = 2, 16, 16
EP1 = E // NS
EPT = E // (NC * NS)
CH = 80
BC = 2000
RPT = NPAD // NS


def _prep_body(x_ref, w_ref, asv_ref, adv_ref,
               xwa_ref, xwb_ref, asrc_ref, adst_ref, m_ref):
    xw = jnp.dot(x_ref[...], w_ref[...], preferred_element_type=jnp.float32)
    xwa_ref[...] = xw[:, :C2]
    xwb_ref[...] = xw[:, C2:]
    a_src = jnp.sum(xw * asv_ref[...], axis=1)
    a_dst = jnp.sum(xw * adv_ref[...], axis=1)
    asrc_ref[...] = a_src[None, :]
    adst_ref[...] = a_dst[None, :]
    m = jnp.max(a_src) + jnp.max(a_dst)
    m = jnp.where(m >= 0.0, m, 0.2 * m)
    m_ref[...] = jnp.full((1, 128), m, jnp.float32)


def _finish_body(p_ref, b_ref, o_ref):
    o_ref[:, :C2] = p_ref[0, 0] + p_ref[1, 0] + b_ref[:, :C2]
    o_ref[:, C2:] = p_ref[0, 1] + p_ref[1, 1] + b_ref[:, C2:]


_sc_mesh = plsc.VectorSubcoreMesh(core_axis_name="c", subcore_axis_name="s")


@functools.partial(
    pl.kernel,
    out_type=jax.ShapeDtypeStruct((NC, 2, NPAD, C2), jnp.float32),
    mesh=_sc_mesh,
    compiler_params=pltpu.CompilerParams(needs_layout_passes=False, use_tc_tiling_on_sc=False),
    scratch_types=[
        pltpu.VMEM((N,), jnp.float32),
        pltpu.VMEM((N,), jnp.float32),
        pltpu.VMEM((1, 128), jnp.float32),
        pltpu.VMEM((BC,), jnp.int32),
        pltpu.VMEM((BC,), jnp.int32),
        pltpu.VMEM((NPAD,), jnp.float32),
        pltpu.VMEM((CH,), jnp.float32),
        pltpu.VMEM((CH,), jnp.int32),
        pltpu.VMEM((CH,), jnp.int32),
        pltpu.VMEM((CH,), jnp.int32),
        pltpu.VMEM((CH,), jnp.int32),
        pltpu.VMEM((CH,), jnp.int32),
        pltpu.VMEM((CH,), jnp.float32),
        pltpu.VMEM((CH, C2), jnp.float32),
        pltpu.VMEM((CH, C2), jnp.float32),
        pltpu.VMEM((RPT,), jnp.float32),
        pltpu.VMEM_SHARED((NPAD,), jnp.float32),
        pltpu.VMEM_SHARED((NPAD, C2), jnp.float32),
        pltpu.SemaphoreType.DMA,
        pltpu.SemaphoreType.DMA,
    ],
)
def _sc_gat(xwa_hbm, xwb_hbm, asrc_hbm, adst_hbm, m_hbm, src_hbm, dst_hbm,
            pout_hbm,
            asrc_v, adst_v, m_v, src_c, dst_c, den_v,
            ee_sm, dst_sm, dst_sma, dst_smb, src_sma, src_smb, al_sm,
            rows_a, rows_b, zb_v, den_sh, out_sh, sem_a, sem_b):
    c = lax.axis_index("c")
    s = lax.axis_index("s")

    pltpu.sync_copy(asrc_hbm.at[0], asrc_v)
    pltpu.sync_copy(adst_hbm.at[0], adst_v)
    pltpu.sync_copy(m_hbm, m_v)
    mvec = m_v[0, pl.ds(0, L)]

    def _zb(i, carry):
        zb_v[pl.ds(i * L, L)] = jnp.zeros((L,), jnp.float32)
        return carry
    lax.fori_loop(0, RPT // L, _zb, 0)

    def _zr(i, carry):
        for q in range(C2 // L):
            rows_a[i, pl.ds(q * L, L)] = jnp.zeros((L,), jnp.float32)
        return carry
    lax.fori_loop(0, CH, _zr, 0)

    pltpu.sync_copy(zb_v, den_sh.at[pl.ds(s * RPT, RPT)])

    def _zo(j, carry):
        pltpu.sync_copy(rows_a, out_sh.at[pl.ds(s * RPT + j * CH, CH)])
        return carry
    lax.fori_loop(0, RPT // CH, _zo, 0)

    plsc.subcore_barrier()

    def _p1(j, carry):
        base = s * EP1 + j * BC
        pltpu.sync_copy(src_hbm.at[pl.ds(base, BC)], src_c)
        pltpu.sync_copy(dst_hbm.at[pl.ds(base, BC)], dst_c)

        def _p1c(t, cc):
            for v in range(CH // L):
                o = t * CH + v * L
                si = src_c[pl.ds(o, L)]
                di = dst_c[pl.ds(o, L)]
                e = plsc.load_gather(asrc_v, [si]) + plsc.load_gather(adst_v, [di])
                e = jnp.where(e >= 0.0, e, 0.2 * e)
                ee_sm[pl.ds(v * L, L)] = jnp.exp(e - mvec)
                dst_sm[pl.ds(v * L, L)] = di
            pltpu.sync_copy(ee_sm, den_sh.at[dst_sm], add=True)
            return cc
        lax.fori_loop(0, BC // CH, _p1c, 0)
        return carry
    lax.fori_loop(0, EP1 // BC, _p1, 0)

    plsc.subcore_barrier()
    pltpu.sync_copy(den_sh, den_v)

    for h, xw_hbm in enumerate((xwa_hbm, xwb_hbm)):
        if h == 1:
            plsc.subcore_barrier()
            def _zr2(i, carry):
                for q in range(C2 // L):
                    rows_a[i, pl.ds(q * L, L)] = jnp.zeros((L,), jnp.float32)
                return carry
            lax.fori_loop(0, CH, _zr2, 0)
            def _zo2(j, carry):
                pltpu.sync_copy(rows_a, out_sh.at[pl.ds(s * RPT + j * CH, CH)])
                return carry
            lax.fori_loop(0, RPT // CH, _zo2, 0)
            plsc.subcore_barrier()

        def _stage(t, sm_s, sm_d):
            for v in range(CH // L):
                o = t * CH + v * L
                sm_s[pl.ds(v * L, L)] = src_c[pl.ds(o, L)]
                sm_d[pl.ds(v * L, L)] = dst_c[pl.ds(o, L)]

        def _fire(sm_s, rows, sem):
            pltpu.async_copy(xw_hbm.at[sm_s], rows, sem)

        def _alpha(sm_s, sm_d):
            for v in range(CH // L):
                si = sm_s[pl.ds(v * L, L)]
                di = sm_d[pl.ds(v * L, L)]
                e = (plsc.load_gather(asrc_v, [si])
                     + plsc.load_gather(adst_v, [di]))
                e = jnp.where(e >= 0.0, e, 0.2 * e)
                ee = jnp.exp(e - mvec)
                den = plsc.load_gather(den_v, [di])
                al_sm[pl.ds(v * L, L)] = ee / (den + 1e-16)

        def _drain_scale_scatter(sm_s, sm_d, rows, sem):
            pltpu.make_async_copy(xw_hbm.at[sm_s], rows, sem).wait()

            def _scale(g, dd):
                av = al_sm[pl.ds(g * L, L)]
                for j in range(L):
                    a = av[j]
                    r = g * L + j
                    for q in range(C2 // L):
                        rows[r, pl.ds(q * L, L)] = rows[r, pl.ds(q * L, L)] * a
                return dd
            lax.fori_loop(0, CH // L, _scale, 0)
            pltpu.sync_copy(rows, out_sh.at[sm_d], add=True)

        def _p2(u, carry):
            base = s * EP1 + c * EPT + u * BC
            pltpu.sync_copy(src_hbm.at[pl.ds(base, BC)], src_c)
            pltpu.sync_copy(dst_hbm.at[pl.ds(base, BC)], dst_c)

            _stage(0, src_sma, dst_sma)
            _fire(src_sma, rows_a, sem_a)

            def _p2c(t, cc):
                _stage(2 * t + 1, src_smb, dst_smb)
                _fire(src_smb, rows_b, sem_b)
                _alpha(src_sma, dst_sma)
                _drain_scale_scatter(src_sma, dst_sma, rows_a, sem_a)
                _stage(2 * t + 2, src_sma, dst_sma)
                _fire(src_sma, rows_a, sem_a)
                _alpha(src_smb, dst_smb)
                _drain_scale_scatter(src_smb, dst_smb, rows_b, sem_b)
                return cc
            lax.fori_loop(0, (BC // CH) // 2, _p2c, 0)

            _alpha(src_sma, dst_sma)
            _drain_scale_scatter(src_sma, dst_sma, rows_a, sem_a)
            return carry
        lax.fori_loop(0, EPT // BC, _p2, 0)

        plsc.subcore_barrier()
        pltpu.sync_copy(out_sh.at[pl.ds(s * RPT, RPT)],
                        pout_hbm.at[c, h, pl.ds(s * RPT, RPT)])


def kernel(x, edge_index, W, att_src, att_dst, bias):
    src = edge_index[0].astype(jnp.int32)
    dst = edge_index[1].astype(jnp.int32)
    asv = att_src.reshape(1, C)
    adv = att_dst.reshape(1, C)

    xwa, xwb, asrc, adst, m = pl.pallas_call(
        _prep_body,
        out_shape=[
            jax.ShapeDtypeStruct((N, C2), jnp.float32),
            jax.ShapeDtypeStruct((N, C2), jnp.float32),
            jax.ShapeDtypeStruct((1, N), jnp.float32),
            jax.ShapeDtypeStruct((1, N), jnp.float32),
            jax.ShapeDtypeStruct((1, 128), jnp.float32),
        ],
    )(x, W, asv, adv)

    pout = _sc_gat(xwa, xwb, asrc, adst, m, src, dst)

    out = pl.pallas_call(
        _finish_body,
        out_shape=jax.ShapeDtypeStruct((NPAD, C), jnp.float32),
    )(pout, bias.reshape(1, C))
    return out[:N]


# speedup vs baseline: 33.9659x; 1.8643x over previous
import functools

import jax
import jax.numpy as jnp
from jax import lax
from jax.experimental import pallas as pl
from jax.experimental.pallas import tpu as pltpu
from jax.experimental.pallas import tpu_sc as plsc

N = 10000
E = 320000
C = 128
C2 = C // 2

NPAD = 10240
NC, NS, L = 2, 16, 16
EP1 = E // NS
EPT = E // (NC * NS)
CH = 80
BC = 2000
RPT = NPAD // NS


def _prep_body(x_ref, w_ref, asv_ref, adv_ref,
               xwa_ref, xwb_ref, asrc_ref, adst_ref, m_ref):
    xw = jnp.dot(x_ref[...], w_ref[...], preferred_element_type=jnp.float32)
    xwa_ref[...] = xw[:, :C2]
    xwb_ref[...] = xw[:, C2:]
    a_src = jnp.sum(xw * asv_ref[...], axis=1)
    a_dst = jnp.sum(xw * adv_ref[...], axis=1)
    asrc_ref[...] = a_src[None, :]
    adst_ref[...] = a_dst[None, :]
    m = jnp.max(a_src) + jnp.max(a_dst)
    m = jnp.where(m >= 0.0, m, 0.2 * m)
    m_ref[...] = jnp.full((1, 128), m, jnp.float32)


def _finish_body(p_ref, b_ref, o_ref):
    o_ref[:, :C2] = p_ref[0, 0] + p_ref[1, 0] + b_ref[:, :C2]
    o_ref[:, C2:] = p_ref[0, 1] + p_ref[1, 1] + b_ref[:, C2:]


_sc_mesh = plsc.VectorSubcoreMesh(core_axis_name="c", subcore_axis_name="s")


@functools.partial(
    pl.kernel,
    out_type=jax.ShapeDtypeStruct((NC, 2, NPAD, C2), jnp.float32),
    mesh=_sc_mesh,
    compiler_params=pltpu.CompilerParams(needs_layout_passes=False, use_tc_tiling_on_sc=False),
    scratch_types=[
        pltpu.VMEM((N,), jnp.float32),
        pltpu.VMEM((N,), jnp.float32),
        pltpu.VMEM((1, 128), jnp.float32),
        pltpu.VMEM((BC,), jnp.int32),
        pltpu.VMEM((BC,), jnp.int32),
        pltpu.VMEM((NPAD,), jnp.float32),
        pltpu.VMEM((CH,), jnp.float32),
        pltpu.VMEM((CH,), jnp.int32),
        pltpu.VMEM((CH,), jnp.int32),
        pltpu.VMEM((CH,), jnp.int32),
        pltpu.VMEM((CH,), jnp.int32),
        pltpu.VMEM((CH,), jnp.int32),
        pltpu.VMEM((CH,), jnp.int32),
        pltpu.VMEM((CH,), jnp.int32),
        pltpu.VMEM((CH,), jnp.float32),
        pltpu.VMEM((CH, C2), jnp.float32),
        pltpu.VMEM((CH, C2), jnp.float32),
        pltpu.VMEM((CH, C2), jnp.float32),
        pltpu.VMEM((CH, C2), jnp.float32),
        pltpu.VMEM((RPT,), jnp.float32),
        pltpu.VMEM_SHARED((NPAD,), jnp.float32),
        pltpu.VMEM_SHARED((NPAD, C2), jnp.float32),
        pltpu.SemaphoreType.DMA,
        pltpu.SemaphoreType.DMA,
        pltpu.SemaphoreType.DMA,
        pltpu.SemaphoreType.DMA,
    ],
)
def _sc_gat(xwa_hbm, xwb_hbm, asrc_hbm, adst_hbm, m_hbm, src_hbm, dst_hbm,
            pout_hbm,
            asrc_v, adst_v, m_v, src_c, dst_c, den_v,
            ee_sm, dst_sm, dst_sma, dst_smb, src_sma, src_smb,
            dst_sca, dst_scb, al_sm,
            rows_a, rows_b, rows_sa, rows_sb, zb_v, den_sh, out_sh,
            sem_a, sem_b, sem_sa, sem_sb):
    c = lax.axis_index("c")
    s = lax.axis_index("s")

    pltpu.sync_copy(asrc_hbm.at[0], asrc_v)
    pltpu.sync_copy(adst_hbm.at[0], adst_v)
    pltpu.sync_copy(m_hbm, m_v)
    mvec = m_v[0, pl.ds(0, L)]

    def _zb(i, carry):
        zb_v[pl.ds(i * L, L)] = jnp.zeros((L,), jnp.float32)
        return carry
    lax.fori_loop(0, RPT // L, _zb, 0)

    def _zr(i, carry):
        for q in range(C2 // L):
            rows_a[i, pl.ds(q * L, L)] = jnp.zeros((L,), jnp.float32)
        return carry
    lax.fori_loop(0, CH, _zr, 0)

    pltpu.sync_copy(zb_v, den_sh.at[pl.ds(s * RPT, RPT)])

    def _zo(j, carry):
        pltpu.sync_copy(rows_a, out_sh.at[pl.ds(s * RPT + j * CH, CH)])
        return carry
    lax.fori_loop(0, RPT // CH, _zo, 0)

    plsc.subcore_barrier()

    def _p1(j, carry):
        base = s * EP1 + j * BC
        pltpu.sync_copy(src_hbm.at[pl.ds(base, BC)], src_c)
        pltpu.sync_copy(dst_hbm.at[pl.ds(base, BC)], dst_c)

        def _p1c(t, cc):
            for v in range(CH // L):
                o = t * CH + v * L
                si = src_c[pl.ds(o, L)]
                di = dst_c[pl.ds(o, L)]
                e = plsc.load_gather(asrc_v, [si]) + plsc.load_gather(adst_v, [di])
                e = jnp.where(e >= 0.0, e, 0.2 * e)
                ee_sm[pl.ds(v * L, L)] = jnp.exp(e - mvec)
                dst_sm[pl.ds(v * L, L)] = di
            pltpu.sync_copy(ee_sm, den_sh.at[dst_sm], add=True)
            return cc
        lax.fori_loop(0, BC // CH, _p1c, 0)
        return carry
    lax.fori_loop(0, EP1 // BC, _p1, 0)

    plsc.subcore_barrier()
    pltpu.sync_copy(den_sh, den_v)

    for h, xw_hbm in enumerate((xwa_hbm, xwb_hbm)):
        if h == 1:
            plsc.subcore_barrier()
            def _zr2(i, carry):
                for q in range(C2 // L):
                    rows_a[i, pl.ds(q * L, L)] = jnp.zeros((L,), jnp.float32)
                return carry
            lax.fori_loop(0, CH, _zr2, 0)
            def _zo2(j, carry):
                pltpu.sync_copy(rows_a, out_sh.at[pl.ds(s * RPT + j * CH, CH)])
                return carry
            lax.fori_loop(0, RPT // CH, _zo2, 0)
            plsc.subcore_barrier()

        def _stage(t, sm_s, sm_d):
            for v in range(CH // L):
                o = t * CH + v * L
                sm_s[pl.ds(v * L, L)] = src_c[pl.ds(o, L)]
                sm_d[pl.ds(v * L, L)] = dst_c[pl.ds(o, L)]

        def _fire(sm_s, rows, sem):
            pltpu.async_copy(xw_hbm.at[sm_s], rows, sem)

        def _alpha(sm_s, sm_d):
            for v in range(CH // L):
                si = sm_s[pl.ds(v * L, L)]
                di = sm_d[pl.ds(v * L, L)]
                e = (plsc.load_gather(asrc_v, [si])
                     + plsc.load_gather(adst_v, [di]))
                e = jnp.where(e >= 0.0, e, 0.2 * e)
                ee = jnp.exp(e - mvec)
                den = plsc.load_gather(den_v, [di])
                al_sm[pl.ds(v * L, L)] = ee / (den + 1e-16)

        def _scale_to(rs, rows):
            @plsc.parallel_loop(0, CH // L, 1)
            def _scale(g):
                av = al_sm[pl.ds(g * L, L)]
                for j in range(L):
                    a = av[j]
                    r = g * L + j
                    for q in range(C2 // L):
                        rs[r, pl.ds(q * L, L)] = rows[r, pl.ds(q * L, L)] * a

        def _fire_sc(rs, sm_d, dst_sc, sem_s):
            for v in range(CH // L):
                dst_sc[pl.ds(v * L, L)] = sm_d[pl.ds(v * L, L)]
            pltpu.async_copy(rs, out_sh.at[dst_sc], sem_s, add=True)

        def _wait_sc(rs, dst_sc, sem_s):
            pltpu.make_async_copy(rs, out_sh.at[dst_sc], sem_s).wait()

        def _p2(u, carry):
            base = s * EP1 + c * EPT + u * BC
            pltpu.sync_copy(src_hbm.at[pl.ds(base, BC)], src_c)
            pltpu.sync_copy(dst_hbm.at[pl.ds(base, BC)], dst_c)

            _stage(0, src_sma, dst_sma)
            _fire(src_sma, rows_a, sem_a)

            def _p2c(t, cc):
                _stage(2 * t + 1, src_smb, dst_smb)
                _fire(src_smb, rows_b, sem_b)
                _alpha(src_sma, dst_sma)
                pltpu.make_async_copy(xw_hbm.at[src_sma], rows_a, sem_a).wait()

                @pl.when(t > 0)
                def _():
                    _wait_sc(rows_sa, dst_sca, sem_sa)
                _scale_to(rows_sa, rows_a)
                _fire_sc(rows_sa, dst_sma, dst_sca, sem_sa)

                _stage(2 * t + 2, src_sma, dst_sma)
                _fire(src_sma, rows_a, sem_a)
                _alpha(src_smb, dst_smb)
                pltpu.make_async_copy(xw_hbm.at[src_smb], rows_b, sem_b).wait()

                @pl.when(t > 0)
                def _():
                    _wait_sc(rows_sb, dst_scb, sem_sb)
                _scale_to(rows_sb, rows_b)
                _fire_sc(rows_sb, dst_smb, dst_scb, sem_sb)
                return cc
            lax.fori_loop(0, (BC // CH) // 2, _p2c, 0)

            _alpha(src_sma, dst_sma)
            pltpu.make_async_copy(xw_hbm.at[src_sma], rows_a, sem_a).wait()
            _wait_sc(rows_sa, dst_sca, sem_sa)
            _scale_to(rows_sa, rows_a)
            _fire_sc(rows_sa, dst_sma, dst_sca, sem_sa)

            _wait_sc(rows_sa, dst_sca, sem_sa)
            _wait_sc(rows_sb, dst_scb, sem_sb)
            return carry
        lax.fori_loop(0, EPT // BC, _p2, 0)

        plsc.subcore_barrier()
        pltpu.sync_copy(out_sh.at[pl.ds(s * RPT, RPT)],
                        pout_hbm.at[c, h, pl.ds(s * RPT, RPT)])


def kernel(x, edge_index, W, att_src, att_dst, bias):
    src = edge_index[0].astype(jnp.int32)
    dst = edge_index[1].astype(jnp.int32)
    asv = att_src.reshape(1, C)
    adv = att_dst.reshape(1, C)

    xwa, xwb, asrc, adst, m = pl.pallas_call(
        _prep_body,
        out_shape=[
            jax.ShapeDtypeStruct((N, C2), jnp.float32),
            jax.ShapeDtypeStruct((N, C2), jnp.float32),
            jax.ShapeDtypeStruct((1, N), jnp.float32),
            jax.ShapeDtypeStruct((1, N), jnp.float32),
            jax.ShapeDtypeStruct((1, 128), jnp.float32),
        ],
    )(x, W, asv, adv)

    pout = _sc_gat(xwa, xwb, asrc, adst, m, src, dst)

    out = pl.pallas_call(
        _finish_body,
        out_shape=jax.ShapeDtypeStruct((NPAD, C), jnp.float32),
    )(pout, bias.reshape(1, C))
    return out[:N]


# speedup vs baseline: 37.2549x; 1.0968x over previous
import functools

import jax
import jax.numpy as jnp
from jax import lax
from jax.experimental import pallas as pl
from jax.experimental.pallas import tpu as pltpu
from jax.experimental.pallas import tpu_sc as plsc

N = 10000
E = 320000
C = 128
C2 = C // 2

NPAD = 10240
NC, NS, L = 2, 16, 16
EPAD = 327680
EP1 = EPAD // NS
EPT = EPAD // (NC * NS)
CH = 128
BC = 2048
NCH = BC // CH
RPT = NPAD // NS


def _prep_body(x_ref, w_ref, asv_ref, adv_ref,
               xwa_ref, xwb_ref, asrc_ref, adst_ref, m_ref):
    xw = jnp.dot(x_ref[...], w_ref[...], preferred_element_type=jnp.float32)
    xwa_ref[...] = xw[:, :C2]
    xwb_ref[...] = xw[:, C2:]
    a_src = jnp.sum(xw * asv_ref[...], axis=1)
    a_dst = jnp.sum(xw * adv_ref[...], axis=1)
    asrc_ref[...] = a_src[None, :]
    adst_ref[...] = a_dst[None, :]
    m = jnp.max(a_src) + jnp.max(a_dst)
    m = jnp.where(m >= 0.0, m, 0.2 * m)
    m_ref[...] = jnp.full((1, 128), m, jnp.float32)


def _finish_body(p_ref, b_ref, o_ref):
    o_ref[:, :C2] = p_ref[0, 0] + p_ref[1, 0] + b_ref[:, :C2]
    o_ref[:, C2:] = p_ref[0, 1] + p_ref[1, 1] + b_ref[:, C2:]


_sc_mesh = plsc.VectorSubcoreMesh(core_axis_name="c", subcore_axis_name="s")


@functools.partial(
    pl.kernel,
    out_type=jax.ShapeDtypeStruct((NC, 2, NPAD, C2), jnp.float32),
    mesh=_sc_mesh,
    compiler_params=pltpu.CompilerParams(needs_layout_passes=False, use_tc_tiling_on_sc=False),
    scratch_types=[
        pltpu.VMEM((N,), jnp.float32),
        pltpu.VMEM((N,), jnp.float32),
        pltpu.VMEM((1, 128), jnp.float32),
        pltpu.VMEM((BC,), jnp.int32),
        pltpu.VMEM((BC,), jnp.int32),
        pltpu.VMEM((NPAD,), jnp.float32),
        pltpu.VMEM((CH,), jnp.float32),
        pltpu.VMEM((CH,), jnp.int32),
        pltpu.VMEM((CH,), jnp.float32),
        pltpu.VMEM((CH,), jnp.int32),
        pltpu.VMEM((CH,), jnp.int32),
        pltpu.VMEM((CH,), jnp.int32),
        pltpu.VMEM((CH,), jnp.int32),
        pltpu.VMEM((CH,), jnp.int32),
        pltpu.VMEM((CH,), jnp.int32),
        pltpu.VMEM((CH,), jnp.int32),
        pltpu.VMEM((CH,), jnp.float32),
        pltpu.VMEM((CH, C2), jnp.float32),
        pltpu.VMEM((CH, C2), jnp.float32),
        pltpu.VMEM((CH, C2), jnp.float32),
        pltpu.VMEM((CH, C2), jnp.float32),
        pltpu.VMEM((RPT,), jnp.float32),
        pltpu.VMEM_SHARED((NPAD,), jnp.float32),
        pltpu.VMEM_SHARED((NPAD, C2), jnp.float32),
        pltpu.SemaphoreType.DMA,
        pltpu.SemaphoreType.DMA,
        pltpu.SemaphoreType.DMA,
        pltpu.SemaphoreType.DMA,
        pltpu.SemaphoreType.DMA,
        pltpu.SemaphoreType.DMA,
    ],
)
def _sc_gat(xwa_hbm, xwb_hbm, asrc_hbm, adst_hbm, m_hbm, src_hbm, dst_hbm,
            pout_hbm,
            asrc_v, adst_v, m_v, src_c, dst_c, den_v,
            ee_sm, dst_sm, ee_p1b, dst_p1b, dst_sma, dst_smb, src_sma, src_smb,
            dst_sca, dst_scb, al_sm,
            rows_a, rows_b, rows_sa, rows_sb, zb_v, den_sh, out_sh,
            sem_a, sem_b, sem_sa, sem_sb, sem_p1a, sem_p1b):
    c = lax.axis_index("c")
    s = lax.axis_index("s")

    pltpu.sync_copy(asrc_hbm.at[0], asrc_v)
    pltpu.sync_copy(adst_hbm.at[0], adst_v)
    pltpu.sync_copy(m_hbm, m_v)
    mvec = m_v[0, pl.ds(0, L)]

    def _zb(i, carry):
        zb_v[pl.ds(i * L, L)] = jnp.zeros((L,), jnp.float32)
        return carry
    lax.fori_loop(0, RPT // L, _zb, 0)

    def _zr(i, carry):
        for q in range(C2 // L):
            rows_a[i, pl.ds(q * L, L)] = jnp.zeros((L,), jnp.float32)
        return carry
    lax.fori_loop(0, CH, _zr, 0)

    pltpu.sync_copy(zb_v, den_sh.at[pl.ds(s * RPT, RPT)])

    def _zo(j, carry):
        pltpu.sync_copy(rows_a, out_sh.at[pl.ds(s * RPT + j * CH, CH)])
        return carry
    lax.fori_loop(0, RPT // CH, _zo, 0)

    plsc.subcore_barrier()

    iota16 = jnp.arange(L, dtype=jnp.int32)

    def _p1comp(ab, o, ee_b, dst_b):
        for v in range(CH // L):
            si = src_c[pl.ds(o + v * L, L)]
            di = dst_c[pl.ds(o + v * L, L)]
            e = plsc.load_gather(asrc_v, [si]) + plsc.load_gather(adst_v, [di])
            e = jnp.where(e >= 0.0, e, 0.2 * e)
            ee = jnp.exp(e - mvec)
            ee = jnp.where(ab + o + v * L + iota16 < E, ee, 0.0)
            ee_b[pl.ds(v * L, L)] = ee
            dst_b[pl.ds(v * L, L)] = di

    def _p1fire(ee_b, dst_b, sem):
        pltpu.async_copy(ee_b, den_sh.at[dst_b], sem, add=True)

    def _p1wait(ee_b, dst_b, sem):
        pltpu.make_async_copy(ee_b, den_sh.at[dst_b], sem).wait()

    def _p1(j, carry):
        base = s * EP1 + j * BC
        pltpu.sync_copy(src_hbm.at[pl.ds(base, BC)], src_c)
        pltpu.sync_copy(dst_hbm.at[pl.ds(base, BC)], dst_c)

        _p1comp(base, 0, ee_sm, dst_sm)
        _p1fire(ee_sm, dst_sm, sem_p1a)

        def _p1c(t, cc):
            @pl.when(t > 0)
            def _():
                _p1wait(ee_p1b, dst_p1b, sem_p1b)
            _p1comp(base, (2 * t + 1) * CH, ee_p1b, dst_p1b)
            _p1fire(ee_p1b, dst_p1b, sem_p1b)
            _p1wait(ee_sm, dst_sm, sem_p1a)
            _p1comp(base, (2 * t + 2) * CH, ee_sm, dst_sm)
            _p1fire(ee_sm, dst_sm, sem_p1a)
            return cc
        lax.fori_loop(0, NCH // 2 - 1, _p1c, 0)

        _p1wait(ee_p1b, dst_p1b, sem_p1b)
        _p1comp(base, (NCH - 1) * CH, ee_p1b, dst_p1b)
        _p1fire(ee_p1b, dst_p1b, sem_p1b)
        _p1wait(ee_sm, dst_sm, sem_p1a)
        _p1wait(ee_p1b, dst_p1b, sem_p1b)
        return carry
    lax.fori_loop(0, EP1 // BC, _p1, 0)

    plsc.subcore_barrier()
    pltpu.sync_copy(den_sh, den_v)

    for h, xw_hbm in enumerate((xwa_hbm, xwb_hbm)):
        if h == 1:
            plsc.subcore_barrier()
            def _zr2(i, carry):
                for q in range(C2 // L):
                    rows_a[i, pl.ds(q * L, L)] = jnp.zeros((L,), jnp.float32)
                return carry
            lax.fori_loop(0, CH, _zr2, 0)
            def _zo2(j, carry):
                pltpu.sync_copy(rows_a, out_sh.at[pl.ds(s * RPT + j * CH, CH)])
                return carry
            lax.fori_loop(0, RPT // CH, _zo2, 0)
            plsc.subcore_barrier()

        def _stage(t, sm_s, sm_d):
            for v in range(CH // L):
                o = t * CH + v * L
                sm_s[pl.ds(v * L, L)] = src_c[pl.ds(o, L)]
                sm_d[pl.ds(v * L, L)] = dst_c[pl.ds(o, L)]

        def _fire(sm_s, rows, sem):
            pltpu.async_copy(xw_hbm.at[sm_s], rows, sem)

        def _alpha(ab, sm_s, sm_d):
            for v in range(CH // L):
                si = sm_s[pl.ds(v * L, L)]
                di = sm_d[pl.ds(v * L, L)]
                e = (plsc.load_gather(asrc_v, [si])
                     + plsc.load_gather(adst_v, [di]))
                e = jnp.where(e >= 0.0, e, 0.2 * e)
                ee = jnp.exp(e - mvec)
                den = plsc.load_gather(den_v, [di])
                al = ee / (den + 1e-16)
                al = jnp.where(ab + v * L + iota16 < E, al, 0.0)
                al_sm[pl.ds(v * L, L)] = al

        def _scale_to(rs, rows):
            @plsc.parallel_loop(0, CH // L, 1)
            def _scale(g):
                av = al_sm[pl.ds(g * L, L)]
                for j in range(L):
                    a = av[j]
                    r = g * L + j
                    for q in range(C2 // L):
                        rs[r, pl.ds(q * L, L)] = rows[r, pl.ds(q * L, L)] * a

        def _fire_sc(rs, sm_d, dst_sc, sem_s):
            for v in range(CH // L):
                dst_sc[pl.ds(v * L, L)] = sm_d[pl.ds(v * L, L)]
            pltpu.async_copy(rs, out_sh.at[dst_sc], sem_s, add=True)

        def _wait_sc(rs, dst_sc, sem_s):
            pltpu.make_async_copy(rs, out_sh.at[dst_sc], sem_s).wait()

        def _p2(u, carry):
            base = s * EP1 + c * EPT + u * BC
            pltpu.sync_copy(src_hbm.at[pl.ds(base, BC)], src_c)
            pltpu.sync_copy(dst_hbm.at[pl.ds(base, BC)], dst_c)

            _stage(0, src_sma, dst_sma)
            _fire(src_sma, rows_a, sem_a)

            def _p2c(t, cc):
                _stage(2 * t + 1, src_smb, dst_smb)
                _fire(src_smb, rows_b, sem_b)
                _alpha(base + 2 * t * CH, src_sma, dst_sma)
                pltpu.make_async_copy(xw_hbm.at[src_sma], rows_a, sem_a).wait()

                @pl.when(t > 0)
                def _():
                    _wait_sc(rows_sa, dst_sca, sem_sa)
                _scale_to(rows_sa, rows_a)
                _fire_sc(rows_sa, dst_sma, dst_sca, sem_sa)

                _stage(2 * t + 2, src_sma, dst_sma)
                _fire(src_sma, rows_a, sem_a)
                _alpha(base + (2 * t + 1) * CH, src_smb, dst_smb)
                pltpu.make_async_copy(xw_hbm.at[src_smb], rows_b, sem_b).wait()

                @pl.when(t > 0)
                def _():
                    _wait_sc(rows_sb, dst_scb, sem_sb)
                _scale_to(rows_sb, rows_b)
                _fire_sc(rows_sb, dst_smb, dst_scb, sem_sb)
                return cc
            lax.fori_loop(0, NCH // 2 - 1, _p2c, 0)

            _stage(NCH - 1, src_smb, dst_smb)
            _fire(src_smb, rows_b, sem_b)
            _alpha(base + (NCH - 2) * CH, src_sma, dst_sma)
            pltpu.make_async_copy(xw_hbm.at[src_sma], rows_a, sem_a).wait()
            _wait_sc(rows_sa, dst_sca, sem_sa)
            _scale_to(rows_sa, rows_a)
            _fire_sc(rows_sa, dst_sma, dst_sca, sem_sa)

            _alpha(base + (NCH - 1) * CH, src_smb, dst_smb)
            pltpu.make_async_copy(xw_hbm.at[src_smb], rows_b, sem_b).wait()
            _wait_sc(rows_sb, dst_scb, sem_sb)
            _scale_to(rows_sb, rows_b)
            _fire_sc(rows_sb, dst_smb, dst_scb, sem_sb)

            _wait_sc(rows_sa, dst_sca, sem_sa)
            _wait_sc(rows_sb, dst_scb, sem_sb)
            return carry
        lax.fori_loop(0, EPT // BC, _p2, 0)

        plsc.subcore_barrier()
        pltpu.sync_copy(out_sh.at[pl.ds(s * RPT, RPT)],
                        pout_hbm.at[c, h, pl.ds(s * RPT, RPT)])


def kernel(x, edge_index, W, att_src, att_dst, bias):
    src = edge_index[0].astype(jnp.int32)
    dst = edge_index[1].astype(jnp.int32)
    pad_idx = (jnp.arange(EPAD - E, dtype=jnp.int32) * 37) % N
    src = jnp.concatenate([src, pad_idx])
    dst = jnp.concatenate([dst, pad_idx])
    asv = att_src.reshape(1, C)
    adv = att_dst.reshape(1, C)

    xwa, xwb, asrc, adst, m = pl.pallas_call(
        _prep_body,
        out_shape=[
            jax.ShapeDtypeStruct((N, C2), jnp.float32),
            jax.ShapeDtypeStruct((N, C2), jnp.float32),
            jax.ShapeDtypeStruct((1, N), jnp.float32),
            jax.ShapeDtypeStruct((1, N), jnp.float32),
            jax.ShapeDtypeStruct((1, 128), jnp.float32),
        ],
    )(x, W, asv, adv)

    pout = _sc_gat(xwa, xwb, asrc, adst, m, src, dst)

    out = pl.pallas_call(
        _finish_body,
        out_shape=jax.ShapeDtypeStruct((NPAD, C), jnp.float32),
    )(pout, bias.reshape(1, C))
    return out[:N]


# speedup vs baseline: 42.6983x; 1.1461x over previous
import functools

import jax
import jax.numpy as jnp
from jax import lax
from jax.experimental import pallas as pl
from jax.experimental.pallas import tpu as pltpu
from jax.experimental.pallas import tpu_sc as plsc

N = 10000
E = 320000
C = 128
C2 = C // 2

NPAD = 10240
NC, NS, L = 2, 16, 16
EPAD = 327680
EP1 = EPAD // NS
EPT = EPAD // (NC * NS)
CH = 128
BC = 2048
NCH = BC // CH
RPT = NPAD // NS


def _prep_body(x_ref, w_ref, asv_ref, adv_ref, ei_ref,
               xwa_ref, xwb_ref, asrc_ref, adst_ref, m_ref, ep_ref):
    xw = jnp.dot(x_ref[...], w_ref[...], preferred_element_type=jnp.float32)
    xwa_ref[...] = xw[:, :C2]
    xwb_ref[...] = xw[:, C2:]
    a_src = jnp.sum(xw * asv_ref[...], axis=1)
    a_dst = jnp.sum(xw * adv_ref[...], axis=1)
    asrc_ref[...] = a_src[None, :]
    adst_ref[...] = a_dst[None, :]
    m = jnp.max(a_src) + jnp.max(a_dst)
    m = jnp.where(m >= 0.0, m, 0.2 * m)
    m_ref[...] = jnp.full((1, 128), m, jnp.float32)
    ep_ref[:, :E] = ei_ref[...]
    pad = jax.lax.broadcasted_iota(jnp.int32, (2, EPAD - E), 1)
    ep_ref[:, E:] = (pad * 37) % N


def _finish_body(p_ref, b_ref, o_ref):
    o_ref[...] = p_ref[0, pl.ds(0, N), :] + p_ref[1, pl.ds(0, N), :] + b_ref[...]


_sc_mesh = plsc.VectorSubcoreMesh(core_axis_name="c", subcore_axis_name="s")


@functools.partial(
    pl.kernel,
    out_type=jax.ShapeDtypeStruct((NC, NPAD, C), jnp.float32),
    mesh=_sc_mesh,
    compiler_params=pltpu.CompilerParams(needs_layout_passes=False, use_tc_tiling_on_sc=False),
    scratch_types=[
        pltpu.VMEM((N,), jnp.float32),
        pltpu.VMEM((N,), jnp.float32),
        pltpu.VMEM((1, 128), jnp.float32),
        pltpu.VMEM((BC,), jnp.int32),
        pltpu.VMEM((BC,), jnp.int32),
        pltpu.VMEM((NPAD,), jnp.float32),
        pltpu.VMEM((CH,), jnp.float32),
        pltpu.VMEM((CH,), jnp.int32),
        pltpu.VMEM((CH,), jnp.float32),
        pltpu.VMEM((CH,), jnp.int32),
        pltpu.VMEM((CH,), jnp.int32),
        pltpu.VMEM((CH,), jnp.int32),
        pltpu.VMEM((CH,), jnp.int32),
        pltpu.VMEM((CH,), jnp.int32),
        pltpu.VMEM((CH,), jnp.int32),
        pltpu.VMEM((CH,), jnp.int32),
        pltpu.VMEM((CH,), jnp.float32),
        pltpu.VMEM((CH, C2), jnp.float32),
        pltpu.VMEM((CH, C2), jnp.float32),
        pltpu.VMEM((CH, C2), jnp.float32),
        pltpu.VMEM((CH, C2), jnp.float32),
        pltpu.VMEM((RPT,), jnp.float32),
        pltpu.VMEM_SHARED((NPAD,), jnp.float32),
        pltpu.VMEM_SHARED((NPAD, C2), jnp.float32),
        pltpu.SemaphoreType.DMA,
        pltpu.SemaphoreType.DMA,
        pltpu.SemaphoreType.DMA,
        pltpu.SemaphoreType.DMA,
        pltpu.SemaphoreType.DMA,
        pltpu.SemaphoreType.DMA,
    ],
)
def _sc_gat(xwa_hbm, xwb_hbm, asrc_hbm, adst_hbm, m_hbm, ep_hbm,
            pout_hbm,
            asrc_v, adst_v, m_v, src_c, dst_c, den_v,
            ee_sm, dst_sm, ee_p1b, dst_p1b, dst_sma, dst_smb, src_sma, src_smb,
            dst_sca, dst_scb, al_sm,
            rows_a, rows_b, rows_sa, rows_sb, zb_v, den_sh, out_sh,
            sem_a, sem_b, sem_sa, sem_sb, sem_p1a, sem_p1b):
    c = lax.axis_index("c")
    s = lax.axis_index("s")

    pltpu.sync_copy(asrc_hbm.at[0], asrc_v)
    pltpu.sync_copy(adst_hbm.at[0], adst_v)
    pltpu.sync_copy(m_hbm, m_v)
    mvec = m_v[0, pl.ds(0, L)]

    def _zb(i, carry):
        zb_v[pl.ds(i * L, L)] = jnp.zeros((L,), jnp.float32)
        return carry
    lax.fori_loop(0, RPT // L, _zb, 0)

    def _zr(i, carry):
        for q in range(C2 // L):
            rows_a[i, pl.ds(q * L, L)] = jnp.zeros((L,), jnp.float32)
        return carry
    lax.fori_loop(0, CH, _zr, 0)

    pltpu.sync_copy(zb_v, den_sh.at[pl.ds(s * RPT, RPT)])

    def _zo(j, carry):
        pltpu.sync_copy(rows_a, out_sh.at[pl.ds(s * RPT + j * CH, CH)])
        return carry
    lax.fori_loop(0, RPT // CH, _zo, 0)

    plsc.subcore_barrier()

    iota16 = jnp.arange(L, dtype=jnp.int32)

    def _p1comp(ab, o, ee_b, dst_b):
        for v in range(CH // L):
            si = src_c[pl.ds(o + v * L, L)]
            di = dst_c[pl.ds(o + v * L, L)]
            e = plsc.load_gather(asrc_v, [si]) + plsc.load_gather(adst_v, [di])
            e = jnp.where(e >= 0.0, e, 0.2 * e)
            ee = jnp.exp(e - mvec)
            ee = jnp.where(ab + o + v * L + iota16 < E, ee, 0.0)
            ee_b[pl.ds(v * L, L)] = ee
            dst_b[pl.ds(v * L, L)] = di

    def _p1fire(ee_b, dst_b, sem):
        pltpu.async_copy(ee_b, den_sh.at[dst_b], sem, add=True)

    def _p1wait(ee_b, dst_b, sem):
        pltpu.make_async_copy(ee_b, den_sh.at[dst_b], sem).wait()

    def _p1(j, carry):
        base = s * EP1 + j * BC
        pltpu.sync_copy(ep_hbm.at[0, pl.ds(base, BC)], src_c)
        pltpu.sync_copy(ep_hbm.at[1, pl.ds(base, BC)], dst_c)

        _p1comp(base, 0, ee_sm, dst_sm)
        _p1fire(ee_sm, dst_sm, sem_p1a)

        def _p1c(t, cc):
            @pl.when(t > 0)
            def _():
                _p1wait(ee_p1b, dst_p1b, sem_p1b)
            _p1comp(base, (2 * t + 1) * CH, ee_p1b, dst_p1b)
            _p1fire(ee_p1b, dst_p1b, sem_p1b)
            _p1wait(ee_sm, dst_sm, sem_p1a)
            _p1comp(base, (2 * t + 2) * CH, ee_sm, dst_sm)
            _p1fire(ee_sm, dst_sm, sem_p1a)
            return cc
        lax.fori_loop(0, NCH // 2 - 1, _p1c, 0)

        _p1wait(ee_p1b, dst_p1b, sem_p1b)
        _p1comp(base, (NCH - 1) * CH, ee_p1b, dst_p1b)
        _p1fire(ee_p1b, dst_p1b, sem_p1b)
        _p1wait(ee_sm, dst_sm, sem_p1a)
        _p1wait(ee_p1b, dst_p1b, sem_p1b)
        return carry
    lax.fori_loop(0, EP1 // BC, _p1, 0)

    plsc.subcore_barrier()
    pltpu.sync_copy(den_sh, den_v)

    for h, xw_hbm in enumerate((xwa_hbm, xwb_hbm)):
        if h == 1:
            plsc.subcore_barrier()
            def _zr2(i, carry):
                for q in range(C2 // L):
                    rows_a[i, pl.ds(q * L, L)] = jnp.zeros((L,), jnp.float32)
                return carry
            lax.fori_loop(0, CH, _zr2, 0)
            def _zo2(j, carry):
                pltpu.sync_copy(rows_a, out_sh.at[pl.ds(s * RPT + j * CH, CH)])
                return carry
            lax.fori_loop(0, RPT // CH, _zo2, 0)
            plsc.subcore_barrier()

        def _stage(t, sm_s, sm_d):
            for v in range(CH // L):
                o = t * CH + v * L
                sm_s[pl.ds(v * L, L)] = src_c[pl.ds(o, L)]
                sm_d[pl.ds(v * L, L)] = dst_c[pl.ds(o, L)]

        def _fire(sm_s, rows, sem):
            pltpu.async_copy(xw_hbm.at[sm_s], rows, sem)

        def _alpha(ab, sm_s, sm_d):
            for v in range(CH // L):
                si = sm_s[pl.ds(v * L, L)]
                di = sm_d[pl.ds(v * L, L)]
                e = (plsc.load_gather(asrc_v, [si])
                     + plsc.load_gather(adst_v, [di]))
                e = jnp.where(e >= 0.0, e, 0.2 * e)
                ee = jnp.exp(e - mvec)
                den = plsc.load_gather(den_v, [di])
                al = ee / (den + 1e-16)
                al = jnp.where(ab + v * L + iota16 < E, al, 0.0)
                al_sm[pl.ds(v * L, L)] = al

        def _scale_to(rs, rows):
            @plsc.parallel_loop(0, CH // L, 1)
            def _scale(g):
                av = al_sm[pl.ds(g * L, L)]
                for j in range(L):
                    a = av[j]
                    r = g * L + j
                    for q in range(C2 // L):
                        rs[r, pl.ds(q * L, L)] = rows[r, pl.ds(q * L, L)] * a

        def _fire_sc(rs, sm_d, dst_sc, sem_s):
            for v in range(CH // L):
                dst_sc[pl.ds(v * L, L)] = sm_d[pl.ds(v * L, L)]
            pltpu.async_copy(rs, out_sh.at[dst_sc], sem_s, add=True)

        def _wait_sc(rs, dst_sc, sem_s):
            pltpu.make_async_copy(rs, out_sh.at[dst_sc], sem_s).wait()

        def _p2(u, carry):
            base = s * EP1 + c * EPT + u * BC
            pltpu.sync_copy(ep_hbm.at[0, pl.ds(base, BC)], src_c)
            pltpu.sync_copy(ep_hbm.at[1, pl.ds(base, BC)], dst_c)

            _stage(0, src_sma, dst_sma)
            _fire(src_sma, rows_a, sem_a)

            def _p2c(t, cc):
                _stage(2 * t + 1, src_smb, dst_smb)
                _fire(src_smb, rows_b, sem_b)
                _alpha(base + 2 * t * CH, src_sma, dst_sma)
                pltpu.make_async_copy(xw_hbm.at[src_sma], rows_a, sem_a).wait()

                @pl.when(t > 0)
                def _():
                    _wait_sc(rows_sa, dst_sca, sem_sa)
                _scale_to(rows_sa, rows_a)
                _fire_sc(rows_sa, dst_sma, dst_sca, sem_sa)

                _stage(2 * t + 2, src_sma, dst_sma)
                _fire(src_sma, rows_a, sem_a)
                _alpha(base + (2 * t + 1) * CH, src_smb, dst_smb)
                pltpu.make_async_copy(xw_hbm.at[src_smb], rows_b, sem_b).wait()

                @pl.when(t > 0)
                def _():
                    _wait_sc(rows_sb, dst_scb, sem_sb)
                _scale_to(rows_sb, rows_b)
                _fire_sc(rows_sb, dst_smb, dst_scb, sem_sb)
                return cc
            lax.fori_loop(0, NCH // 2 - 1, _p2c, 0)

            _stage(NCH - 1, src_smb, dst_smb)
            _fire(src_smb, rows_b, sem_b)
            _alpha(base + (NCH - 2) * CH, src_sma, dst_sma)
            pltpu.make_async_copy(xw_hbm.at[src_sma], rows_a, sem_a).wait()
            _wait_sc(rows_sa, dst_sca, sem_sa)
            _scale_to(rows_sa, rows_a)
            _fire_sc(rows_sa, dst_sma, dst_sca, sem_sa)

            _alpha(base + (NCH - 1) * CH, src_smb, dst_smb)
            pltpu.make_async_copy(xw_hbm.at[src_smb], rows_b, sem_b).wait()
            _wait_sc(rows_sb, dst_scb, sem_sb)
            _scale_to(rows_sb, rows_b)
            _fire_sc(rows_sb, dst_smb, dst_scb, sem_sb)

            _wait_sc(rows_sa, dst_sca, sem_sa)
            _wait_sc(rows_sb, dst_scb, sem_sb)
            return carry
        lax.fori_loop(0, EPT // BC, _p2, 0)

        plsc.subcore_barrier()
        pltpu.sync_copy(out_sh.at[pl.ds(s * RPT, RPT)],
                        pout_hbm.at[c, pl.ds(s * RPT, RPT),
                                    pl.ds(h * C2, C2)])


def kernel(x, edge_index, W, att_src, att_dst, bias):
    asv = att_src.reshape(1, C)
    adv = att_dst.reshape(1, C)

    xwa, xwb, asrc, adst, m, ep = pl.pallas_call(
        _prep_body,
        out_shape=[
            jax.ShapeDtypeStruct((N, C2), jnp.float32),
            jax.ShapeDtypeStruct((N, C2), jnp.float32),
            jax.ShapeDtypeStruct((1, N), jnp.float32),
            jax.ShapeDtypeStruct((1, N), jnp.float32),
            jax.ShapeDtypeStruct((1, 128), jnp.float32),
            jax.ShapeDtypeStruct((2, EPAD), jnp.int32),
        ],
    )(x, W, asv, adv, edge_index.astype(jnp.int32))

    pout = _sc_gat(xwa, xwb, asrc, adst, m, ep)

    out = pl.pallas_call(
        _finish_body,
        out_shape=jax.ShapeDtypeStruct((N, C), jnp.float32),
    )(pout, bias.reshape(1, C))
    return out
